# Initial kernel scaffold; baseline (speedup 1.0000x reference)
import jax
import jax.numpy as jnp
from jax.experimental import pallas as pl


def kernel(x, edge_index, edge_attr, u, batch, e1_1_W, e1_1_b, e1_2_W, e1_2_b, n1_1_W, n1_1_b, n1_2_W, n1_2_b, n1_3_W, n1_3_b, n1_4_W, n1_4_b, g1_1_W, g1_1_b, g1_2_W, g1_2_b, e2_1_W, e2_1_b, e2_2_W, e2_2_b, n2_1_W, n2_1_b, n2_2_W, n2_2_b, n2_3_W, n2_3_b, n2_4_W, n2_4_b, g2_1_W, g2_1_b, g2_2_W, g2_2_b, cls_W, cls_b):
    raise NotImplementedError("write your pallas kernel here")


# speedup vs baseline: 4.9343x; 4.9343x over previous
import functools

import jax
import jax.numpy as jnp
import numpy as np
from jax import lax
from jax.experimental import pallas as pl
from jax.experimental.pallas import tpu as pltpu
from jax.experimental.pallas import tpu_sc as plsc

_BN = float(1.0 / np.sqrt(1.0 + 1e-5))
_F32 = jnp.float32

_N = 10000
_E = 320000
_G = 64
_H = 32

_NBLK = 1000
_EBLK = 4000
_CH = 128
_NW = 32
_NCHUNK = _E // _CH
_TW = 128
_MW = 64


def _dotT(a, b):
    return jnp.dot(a, b, preferred_element_type=_F32)


def _onehot_from(batchf_blk, g):
    nb = batchf_blk.shape[-1]
    io = lax.broadcasted_iota(jnp.int32, (g, nb), 0).astype(_F32)
    return (batchf_blk == io).astype(_F32)


def _proj1_body(batchf, x, u, wxr, wxc, wmx, wu, be1, bn1, t_o):
    oh = _onehot_from(batchf[0], _G)
    up = _dotT(u[...], wu[...])
    upn = lax.dot_general(oh, up, (((0,), (0,)), ((), ())),
                          preferred_element_type=_F32)
    xb = x[...]
    a = _dotT(xb, wxr[...]) + upn + be1[...]
    c = _dotT(xb, wmx[...]) + bn1[...]
    b = _dotT(xb, wxc[...])
    t_o[...] = jnp.concatenate(
        [a, c, b, jnp.zeros((a.shape[0], _TW - 3 * _H), _F32)], axis=1)


def _proj2_body(batchf, x, u_prev, nacc, g1u, g1n, gb1, g2w, gb2,
                wxr, wxc, wmx, wu, be1, bn1, t_o, u2_o):
    acc = nacc[...]
    cnt = jnp.maximum(acc[:, _H:_H + 1], 1.0)
    nagg = acc[:, :_H] / cnt
    hg = jnp.maximum(_dotT(u_prev[...], g1u[...]) + _dotT(nagg, g1n[...])
                     + gb1[...], 0.0)
    u2 = _dotT(hg, g2w[...]) + gb2[...]
    i = pl.program_id(0)

    @pl.when(i == 0)
    def _():
        u2_o[...] = u2

    oh = _onehot_from(batchf[0], _G)
    up = _dotT(u2, wu[...])
    upn = lax.dot_general(oh, up, (((0,), (0,)), ((), ())),
                          preferred_element_type=_F32)
    xb = x[...]
    a = _dotT(xb, wxr[...]) + upn + be1[...]
    c = _dotT(xb, wmx[...]) + bn1[...]
    b = _dotT(xb, wxc[...])
    t_o[...] = jnp.concatenate(
        [a, c, b, jnp.zeros((a.shape[0], _TW - 3 * _H), _F32)], axis=1)


def _edge_body(pad_ones, rg, cg, ea, wea, we2, be2, wme, wm2, bm2,
               ea_o, m_o):
    rgb = rg[...]
    bcol = cg[...][:, 2 * _H:3 * _H]
    h1 = jnp.maximum(rgb[:, :_H] + bcol + _dotT(ea[...], wea[...]), 0.0)
    ea_new = _dotT(h1, we2[...]) + be2[...]
    h2 = jnp.maximum(rgb[:, _H:2 * _H] + _dotT(ea_new, wme[...]), 0.0)
    m = _dotT(h2, wm2[...]) + bm2[...]
    ea_o[...] = ea_new
    nrow = m.shape[0]
    if pad_ones:
        m_o[...] = jnp.concatenate(
            [m, jnp.ones((nrow, 8), _F32),
             jnp.zeros((nrow, _MW - _H - 8), _F32)], axis=1)
    else:
        m_o[...] = jnp.concatenate(
            [m, jnp.zeros((nrow, _MW - _H), _F32)], axis=1)


def _node1_body(batchf, x, p0, p1, u, w3x, w3a, w3u, b3, w4, b4,
                x2_o, nacc_o, rcp_o):
    s = p0[...] + p1[...]
    rc = 1.0 / jnp.maximum(s[:, _H:_H + 1], 1.0)
    agg = s[:, :_H] * rc
    oh = _onehot_from(batchf[0], _G)
    upn = lax.dot_general(oh, _dotT(u[...], w3u[...]),
                          (((0,), (0,)), ((), ())), preferred_element_type=_F32)
    h = jnp.maximum(_dotT(x[...], w3x[...]) + _dotT(agg, w3a[...]) + upn
                    + b3[...], 0.0)
    x2 = _dotT(h, w4[...]) + b4[...]
    x2_o[...] = x2
    rcp_o[...] = jnp.broadcast_to(rc, (rc.shape[0], 8))
    contrib = jnp.concatenate(
        [x2, jnp.ones((x2.shape[0], 8), _F32)], axis=1)
    i = pl.program_id(0)

    @pl.when(i == 0)
    def _():
        nacc_o[...] = jnp.zeros_like(nacc_o)

    nacc_o[...] += _dotT(oh, contrib)


def _node2_body(batchf, x, p0, p1, rcp, u, w3x, w3a, w3u, b3, w4, b4,
                clsw, clsb, x2_o, out_o, nacc_o):
    s = p0[...] + p1[...]
    agg = s[:, :_H] * rcp[...][:, 0:1]
    oh = _onehot_from(batchf[0], _G)
    upn = lax.dot_general(oh, _dotT(u[...], w3u[...]),
                          (((0,), (0,)), ((), ())), preferred_element_type=_F32)
    h = jnp.maximum(_dotT(x[...], w3x[...]) + _dotT(agg, w3a[...]) + upn
                    + b3[...], 0.0)
    x2 = _dotT(h, w4[...]) + b4[...]
    x2_o[...] = x2
    z = _dotT(x2, clsw[...]) + clsb[...]
    z = z - jnp.max(z, axis=1, keepdims=True)
    out_o[...] = z - jnp.log(jnp.sum(jnp.exp(z), axis=1, keepdims=True))
    contrib = jnp.concatenate(
        [x2, jnp.ones((x2.shape[0], 8), _F32)], axis=1)
    i = pl.program_id(0)

    @pl.when(i == 0)
    def _():
        nacc_o[...] = jnp.zeros_like(nacc_o)

    nacc_o[...] += _dotT(oh, contrib)


def _gfinal_body(u2, nacc, g1u, g1n, gb1, g2w, gb2, u3_o):
    acc = nacc[...]
    cnt = jnp.maximum(acc[:, _H:_H + 1], 1.0)
    nagg = acc[:, :_H] / cnt
    hg = jnp.maximum(_dotT(u2[...], g1u[...]) + _dotT(nagg, g1n[...])
                     + gb1[...], 0.0)
    u3_o[...] = _dotT(hg, g2w[...]) + gb2[...]


def _mesh():
    return plsc.VectorSubcoreMesh(core_axis_name="c", subcore_axis_name="s")


def _sc_gather(row_hbm, col_hbm, t_hbm, rg_o, cg_o,
               idx1, idx2, buf1, buf2, sem1, sem2):
    cid = lax.axis_index("c")
    sid = lax.axis_index("s")
    wid = sid * 2 + cid

    def body(k, carry):
        chunk = k * _NW + wid

        @pl.when(chunk < _NCHUNK)
        def _do():
            off = chunk * _CH
            pltpu.sync_copy(row_hbm.at[pl.ds(off, _CH)], idx1)
            pltpu.sync_copy(col_hbm.at[pl.ds(off, _CH)], idx2)
            cp1 = pltpu.async_copy(t_hbm.at[idx1], buf1, sem1)
            cp2 = pltpu.async_copy(t_hbm.at[idx2], buf2, sem2)
            cp1.wait()
            cp2.wait()
            pltpu.sync_copy(buf1, rg_o.at[pl.ds(off, _CH)])
            pltpu.sync_copy(buf2, cg_o.at[pl.ds(off, _CH)])

        return carry

    lax.fori_loop(0, (_NCHUNK + _NW - 1) // _NW, body, 0)


def _sc_scatter(m_hbm, col_hbm, zeros_hbm, p0_o, p1_o, idx, val, acc, sem):
    cid = lax.axis_index("c")
    sid = lax.axis_index("s")
    wid = sid * 2 + cid

    @pl.when(sid == 0)
    def _init():
        pltpu.sync_copy(zeros_hbm, acc)

    plsc.subcore_barrier()

    def body(k, carry):
        chunk = k * _NW + wid

        @pl.when(chunk < _NCHUNK)
        def _do():
            off = chunk * _CH
            pltpu.sync_copy(col_hbm.at[pl.ds(off, _CH)], idx)
            pltpu.sync_copy(m_hbm.at[pl.ds(off, _CH)], val)
            pltpu.sync_copy(val, acc.at[idx], add=True)

        return carry

    lax.fori_loop(0, (_NCHUNK + _NW - 1) // _NW, body, 0)
    plsc.subcore_barrier()

    @pl.when((sid == 0) & (cid == 0))
    def _flush0():
        pltpu.sync_copy(acc, p0_o)

    @pl.when((sid == 0) & (cid == 1))
    def _flush1():
        pltpu.sync_copy(acc, p1_o)


def _make_sc_gather():
    return functools.partial(
        pl.kernel,
        _sc_gather,
        mesh=_mesh(),
        out_type=[jax.ShapeDtypeStruct((_E, _TW), _F32),
                  jax.ShapeDtypeStruct((_E, _TW), _F32)],
        scratch_types=[pltpu.VMEM((_CH,), jnp.int32),
                       pltpu.VMEM((_CH,), jnp.int32),
                       pltpu.VMEM((_CH, _TW), _F32),
                       pltpu.VMEM((_CH, _TW), _F32),
                       pltpu.SemaphoreType.DMA,
                       pltpu.SemaphoreType.DMA],
    )()


def _make_sc_scatter():
    return functools.partial(
        pl.kernel,
        _sc_scatter,
        mesh=_mesh(),
        out_type=[jax.ShapeDtypeStruct((_N, _MW), _F32),
                  jax.ShapeDtypeStruct((_N, _MW), _F32)],
        scratch_types=[pltpu.VMEM((_CH,), jnp.int32),
                       pltpu.VMEM((_CH, _MW), _F32),
                       pltpu.VMEM_SHARED((_N, _MW), _F32),
                       pltpu.SemaphoreType.DMA],
    )()


def _full(shape):
    nd = len(shape)
    return pl.BlockSpec(shape, lambda i: (0,) * nd)


def kernel(x, edge_index, edge_attr, u, batch,
           e1_1_W, e1_1_b, e1_2_W, e1_2_b,
           n1_1_W, n1_1_b, n1_2_W, n1_2_b,
           n1_3_W, n1_3_b, n1_4_W, n1_4_b,
           g1_1_W, g1_1_b, g1_2_W, g1_2_b,
           e2_1_W, e2_1_b, e2_2_W, e2_2_b,
           n2_1_W, n2_1_b, n2_2_W, n2_2_b,
           n2_3_W, n2_3_b, n2_4_W, n2_4_b,
           g2_1_W, g2_1_b, g2_2_W, g2_2_b,
           cls_W, cls_b):
    n, d = x.shape
    e = edge_index.shape[1]
    g, dg = u.shape
    de = edge_attr.shape[1]
    assert (n, e, g) == (_N, _E, _G)

    row = edge_index[0]
    col = edge_index[1]
    nb_grid = n // _NBLK
    eb_grid = e // _EBLK
    batchf = batch.astype(_F32).reshape(nb_grid, 1, _NBLK)
    zeros_acc = jnp.zeros((_N, _MW), _F32)

    def rT(w):
        return jnp.transpose(w)

    def b2d(b):
        return b.reshape(1, -1)

    w1 = e1_1_W
    t1 = pl.pallas_call(
        _proj1_body,
        grid=(nb_grid,),
        in_specs=[
            pl.BlockSpec((1, 1, _NBLK), lambda i: (i, 0, 0)),
            pl.BlockSpec((_NBLK, d), lambda i: (i, 0)),
            _full((g, dg)),
            _full((d, _H)), _full((d, _H)), _full((d, _H)),
            _full((dg, _H)),
            _full((1, _H)), _full((1, _H)),
        ],
        out_specs=pl.BlockSpec((_NBLK, _TW), lambda i: (i, 0)),
        out_shape=jax.ShapeDtypeStruct((n, _TW), _F32),
    )(batchf, x, u,
      rT(w1[:, :d]), rT(w1[:, d:2 * d]), rT(n1_1_W[:, :d]),
      rT(w1[:, 2 * d + de:]),
      b2d(e1_1_b), b2d(n1_1_b))

    rg1, cg1 = _make_sc_gather()(row, col, t1)

    ea1, m1 = pl.pallas_call(
        functools.partial(_edge_body, True),
        grid=(eb_grid,),
        in_specs=[
            pl.BlockSpec((_EBLK, _TW), lambda i: (i, 0)),
            pl.BlockSpec((_EBLK, _TW), lambda i: (i, 0)),
            pl.BlockSpec((_EBLK, de), lambda i: (i, 0)),
            _full((de, _H)), _full((_H, _H)), _full((1, _H)),
            _full((_H, _H)), _full((_H, _H)), _full((1, _H)),
        ],
        out_specs=[
            pl.BlockSpec((_EBLK, _H), lambda i: (i, 0)),
            pl.BlockSpec((_EBLK, _MW), lambda i: (i, 0)),
        ],
        out_shape=[jax.ShapeDtypeStruct((e, _H), _F32),
                   jax.ShapeDtypeStruct((e, _MW), _F32)],
    )(rg1, cg1, edge_attr,
      rT(e1_1_W[:, 2 * d:2 * d + de]), rT(e1_2_W) * _BN, b2d(e1_2_b),
      rT(n1_1_W[:, d:]), rT(n1_2_W) * _BN, b2d(n1_2_b))

    p0, p1 = _make_sc_scatter()(m1, col, zeros_acc)

    x2, nacc1, rcp = pl.pallas_call(
        _node1_body,
        grid=(nb_grid,),
        in_specs=[
            pl.BlockSpec((1, 1, _NBLK), lambda i: (i, 0, 0)),
            pl.BlockSpec((_NBLK, d), lambda i: (i, 0)),
            pl.BlockSpec((_NBLK, _MW), lambda i: (i, 0)),
            pl.BlockSpec((_NBLK, _MW), lambda i: (i, 0)),
            _full((g, dg)),
            _full((d, _H)), _full((_H, _H)), _full((dg, _H)),
            _full((1, _H)), _full((_H, _H)), _full((1, _H)),
        ],
        out_specs=[
            pl.BlockSpec((_NBLK, _H), lambda i: (i, 0)),
            _full((g, _H + 8)),
            pl.BlockSpec((_NBLK, 8), lambda i: (i, 0)),
        ],
        out_shape=[jax.ShapeDtypeStruct((n, _H), _F32),
                   jax.ShapeDtypeStruct((g, _H + 8), _F32),
                   jax.ShapeDtypeStruct((n, 8), _F32)],
    )(batchf, x, p0, p1, u,
      rT(n1_3_W[:, :d]), rT(n1_3_W[:, d:d + _H]), rT(n1_3_W[:, d + _H:]),
      b2d(n1_3_b), rT(n1_4_W) * _BN, b2d(n1_4_b))

    w2 = e2_1_W
    t2, u2 = pl.pallas_call(
        _proj2_body,
        grid=(nb_grid,),
        in_specs=[
            pl.BlockSpec((1, 1, _NBLK), lambda i: (i, 0, 0)),
            pl.BlockSpec((_NBLK, _H), lambda i: (i, 0)),
            _full((g, dg)),
            _full((g, _H + 8)),
            _full((dg, _H)), _full((_H, _H)), _full((1, _H)),
            _full((_H, _H)), _full((1, _H)),
            _full((_H, _H)), _full((_H, _H)), _full((_H, _H)),
            _full((_H, _H)),
            _full((1, _H)), _full((1, _H)),
        ],
        out_specs=[
            pl.BlockSpec((_NBLK, _TW), lambda i: (i, 0)),
            _full((g, _H)),
        ],
        out_shape=[jax.ShapeDtypeStruct((n, _TW), _F32),
                   jax.ShapeDtypeStruct((g, _H), _F32)],
    )(batchf, x2, u, nacc1,
      rT(g1_1_W[:, :dg]), rT(g1_1_W[:, dg:]), b2d(g1_1_b),
      rT(g1_2_W), b2d(g1_2_b),
      rT(w2[:, :_H]), rT(w2[:, _H:2 * _H]), rT(n2_1_W[:, :_H]),
      rT(w2[:, 3 * _H:]),
      b2d(e2_1_b), b2d(n2_1_b))

    rg2, cg2 = _make_sc_gather()(row, col, t2)

    ea2, m2 = pl.pallas_call(
        functools.partial(_edge_body, False),
        grid=(eb_grid,),
        in_specs=[
            pl.BlockSpec((_EBLK, _TW), lambda i: (i, 0)),
            pl.BlockSpec((_EBLK, _TW), lambda i: (i, 0)),
            pl.BlockSpec((_EBLK, _H), lambda i: (i, 0)),
            _full((_H, _H)), _full((_H, _H)), _full((1, _H)),
            _full((_H, _H)), _full((_H, _H)), _full((1, _H)),
        ],
        out_specs=[
            pl.BlockSpec((_EBLK, _H), lambda i: (i, 0)),
            pl.BlockSpec((_EBLK, _MW), lambda i: (i, 0)),
        ],
        out_shape=[jax.ShapeDtypeStruct((e, _H), _F32),
                   jax.ShapeDtypeStruct((e, _MW), _F32)],
    )(rg2, cg2, ea1,
      rT(e2_1_W[:, 2 * _H:3 * _H]), rT(e2_2_W) * _BN, b2d(e2_2_b),
      rT(n2_1_W[:, _H:]), rT(n2_2_W) * _BN, b2d(n2_2_b))

    q0, q1 = _make_sc_scatter()(m2, col, zeros_acc)

    x3, out, nacc2 = pl.pallas_call(
        _node2_body,
        grid=(nb_grid,),
        in_specs=[
            pl.BlockSpec((1, 1, _NBLK), lambda i: (i, 0, 0)),
            pl.BlockSpec((_NBLK, _H), lambda i: (i, 0)),
            pl.BlockSpec((_NBLK, _MW), lambda i: (i, 0)),
            pl.BlockSpec((_NBLK, _MW), lambda i: (i, 0)),
            pl.BlockSpec((_NBLK, 8), lambda i: (i, 0)),
            _full((g, _H)),
            _full((_H, _H)), _full((_H, _H)), _full((_H, _H)),
            _full((1, _H)), _full((_H, _H)), _full((1, _H)),
            _full((_H, 20)), _full((1, 20)),
        ],
        out_specs=[
            pl.BlockSpec((_NBLK, _H), lambda i: (i, 0)),
            pl.BlockSpec((_NBLK, 20), lambda i: (i, 0)),
            _full((g, _H + 8)),
        ],
        out_shape=[jax.ShapeDtypeStruct((n, _H), _F32),
                   jax.ShapeDtypeStruct((n, 20), _F32),
                   jax.ShapeDtypeStruct((g, _H + 8), _F32)],
    )(batchf, x2, q0, q1, rcp, u2,
      rT(n2_3_W[:, :_H]), rT(n2_3_W[:, _H:2 * _H]), rT(n2_3_W[:, 2 * _H:]),
      b2d(n2_3_b), rT(n2_4_W) * _BN, b2d(n2_4_b),
      rT(cls_W), b2d(cls_b))

    u3 = pl.pallas_call(
        _gfinal_body,
        grid=(1,),
        in_specs=[
            _full((g, _H)), _full((g, _H + 8)),
            _full((_H, _H)), _full((_H, _H)), _full((1, _H)),
            _full((_H, _H)), _full((1, _H)),
        ],
        out_specs=_full((g, _H)),
        out_shape=jax.ShapeDtypeStruct((g, _H), _F32),
    )(u2, nacc2,
      rT(g2_1_W[:, :_H]), rT(g2_1_W[:, _H:]), b2d(g2_1_b),
      rT(g2_2_W), b2d(g2_2_b))

    return (out, x3, ea2, u3)


# speedup vs baseline: 5.0322x; 1.0198x over previous
import functools

import jax
import jax.numpy as jnp
import numpy as np
from jax import lax
from jax.experimental import pallas as pl
from jax.experimental.pallas import tpu as pltpu
from jax.experimental.pallas import tpu_sc as plsc

_BN = float(1.0 / np.sqrt(1.0 + 1e-5))
_F32 = jnp.float32

_N = 10000
_E = 320000
_G = 64
_H = 32

_NBLK = 1000
_EBLK = 10000
_CH = 128
_NW = 32
_NCHUNK = _E // _CH
_TW = 128
_MW = 40
_MW2 = 32


def _dotT(a, b):
    return jnp.dot(a, b, preferred_element_type=_F32)


def _onehot_from(batchf_blk, g):
    nb = batchf_blk.shape[-1]
    io = lax.broadcasted_iota(jnp.int32, (g, nb), 0).astype(_F32)
    return (batchf_blk == io).astype(_F32)


def _proj1_body(batchf, x, u, wxr, wxc, wmx, wu, be1, bn1, t_o):
    oh = _onehot_from(batchf[0], _G)
    up = _dotT(u[...], wu[...])
    upn = lax.dot_general(oh, up, (((0,), (0,)), ((), ())),
                          preferred_element_type=_F32)
    xb = x[...]
    a = _dotT(xb, wxr[...]) + upn + be1[...]
    c = _dotT(xb, wmx[...]) + bn1[...]
    b = _dotT(xb, wxc[...])
    t_o[...] = jnp.concatenate(
        [a, c, b, jnp.zeros((a.shape[0], _TW - 3 * _H), _F32)], axis=1)


def _proj2_body(batchf, x, u_prev, nacc, g1u, g1n, gb1, g2w, gb2,
                wxr, wxc, wmx, wu, be1, bn1, t_o, u2_o):
    acc = nacc[...]
    cnt = jnp.maximum(acc[:, _H:_H + 1], 1.0)
    nagg = acc[:, :_H] / cnt
    hg = jnp.maximum(_dotT(u_prev[...], g1u[...]) + _dotT(nagg, g1n[...])
                     + gb1[...], 0.0)
    u2 = _dotT(hg, g2w[...]) + gb2[...]
    i = pl.program_id(0)

    @pl.when(i == 0)
    def _():
        u2_o[...] = u2

    oh = _onehot_from(batchf[0], _G)
    up = _dotT(u2, wu[...])
    upn = lax.dot_general(oh, up, (((0,), (0,)), ((), ())),
                          preferred_element_type=_F32)
    xb = x[...]
    a = _dotT(xb, wxr[...]) + upn + be1[...]
    c = _dotT(xb, wmx[...]) + bn1[...]
    b = _dotT(xb, wxc[...])
    t_o[...] = jnp.concatenate(
        [a, c, b, jnp.zeros((a.shape[0], _TW - 3 * _H), _F32)], axis=1)


def _edge_body(pad_ones, gr, gc, ea, wea, wcomb, be2, bwme, wm2, bm2,
               ea_o, m_o):
    grb = gr[...]
    gcb = gc[...]
    h1 = jnp.maximum(grb[:, :_H] + gcb[:, 2 * _H:3 * _H]
                     + _dotT(ea[...], wea[...]), 0.0)
    t = _dotT(h1, wcomb[...])
    ea_new = t[:, :_H] + be2[...]
    h2 = jnp.maximum(grb[:, _H:2 * _H] + t[:, _H:] + bwme[...], 0.0)
    m = _dotT(h2, wm2[...]) + bm2[...]
    ea_o[...] = ea_new
    nb = m.shape[0]
    if pad_ones:
        m_o[...] = jnp.concatenate([m, jnp.ones((nb, 8), _F32)], axis=1)
    else:
        m_o[...] = m


def _node1_body(batchf, x, p0, p1, u, w3x, w3a, w3u, b3, w4, b4,
                x2_o, nacc_o, rcp_o):
    s = p0[...] + p1[...]
    rc = 1.0 / jnp.maximum(s[:, _H:_H + 1], 1.0)
    agg = s[:, :_H] * rc
    oh = _onehot_from(batchf[0], _G)
    upn = lax.dot_general(oh, _dotT(u[...], w3u[...]),
                          (((0,), (0,)), ((), ())), preferred_element_type=_F32)
    h = jnp.maximum(_dotT(x[...], w3x[...]) + _dotT(agg, w3a[...]) + upn
                    + b3[...], 0.0)
    x2 = _dotT(h, w4[...]) + b4[...]
    x2_o[...] = x2
    rcp_o[...] = jnp.broadcast_to(rc, (rc.shape[0], 8))
    contrib = jnp.concatenate(
        [x2, jnp.ones((x2.shape[0], 8), _F32)], axis=1)
    i = pl.program_id(0)

    @pl.when(i == 0)
    def _():
        nacc_o[...] = jnp.zeros_like(nacc_o)

    nacc_o[...] += _dotT(oh, contrib)


def _node2_body(batchf, x, p0, p1, rcp, u, w3x, w3a, w3u, b3, w4, b4,
                clsw, clsb, x2_o, out_o, nacc_o):
    s = p0[...] + p1[...]
    agg = s[:, :_H] * rcp[...][:, 0:1]
    oh = _onehot_from(batchf[0], _G)
    upn = lax.dot_general(oh, _dotT(u[...], w3u[...]),
                          (((0,), (0,)), ((), ())), preferred_element_type=_F32)
    h = jnp.maximum(_dotT(x[...], w3x[...]) + _dotT(agg, w3a[...]) + upn
                    + b3[...], 0.0)
    x2 = _dotT(h, w4[...]) + b4[...]
    x2_o[...] = x2
    z = _dotT(x2, clsw[...]) + clsb[...]
    z = z - jnp.max(z, axis=1, keepdims=True)
    out_o[...] = z - jnp.log(jnp.sum(jnp.exp(z), axis=1, keepdims=True))
    contrib = jnp.concatenate(
        [x2, jnp.ones((x2.shape[0], 8), _F32)], axis=1)
    i = pl.program_id(0)

    @pl.when(i == 0)
    def _():
        nacc_o[...] = jnp.zeros_like(nacc_o)

    nacc_o[...] += _dotT(oh, contrib)


def _gfinal_body(u2, nacc, g1u, g1n, gb1, g2w, gb2, u3_o):
    acc = nacc[...]
    cnt = jnp.maximum(acc[:, _H:_H + 1], 1.0)
    nagg = acc[:, :_H] / cnt
    hg = jnp.maximum(_dotT(u2[...], g1u[...]) + _dotT(nagg, g1n[...])
                     + gb1[...], 0.0)
    u3_o[...] = _dotT(hg, g2w[...]) + gb2[...]


def _mesh():
    return plsc.VectorSubcoreMesh(core_axis_name="c", subcore_axis_name="s")


def _sc_gather(row_hbm, col_hbm, t_hbm, gr_o, gc_o,
               idx1, idx2, buf1, buf2, sem1, sem2):
    cid = lax.axis_index("c")
    sid = lax.axis_index("s")
    wid = sid * 2 + cid

    def body(k, carry):
        chunk = k * _NW + wid

        @pl.when(chunk < _NCHUNK)
        def _do():
            off = chunk * _CH
            pltpu.sync_copy(row_hbm.at[pl.ds(off, _CH)], idx1)
            pltpu.sync_copy(col_hbm.at[pl.ds(off, _CH)], idx2)
            cp1 = pltpu.async_copy(t_hbm.at[idx1], buf1, sem1)
            cp2 = pltpu.async_copy(t_hbm.at[idx2], buf2, sem2)
            cp1.wait()
            cp2.wait()
            pltpu.sync_copy(buf1, gr_o.at[pl.ds(off, _CH)])
            pltpu.sync_copy(buf2, gc_o.at[pl.ds(off, _CH)])

        return carry

    lax.fori_loop(0, (_NCHUNK + _NW - 1) // _NW, body, 0)


def _sc_scatter(m_hbm, col_hbm, zeros_hbm, p0_o, p1_o, idx, val, acc, sem):
    cid = lax.axis_index("c")
    sid = lax.axis_index("s")
    wid = sid * 2 + cid

    @pl.when(sid == 0)
    def _init():
        pltpu.sync_copy(zeros_hbm, acc)

    plsc.subcore_barrier()

    def body(k, carry):
        chunk = k * _NW + wid

        @pl.when(chunk < _NCHUNK)
        def _do():
            off = chunk * _CH
            w = val.shape[1]
            pltpu.sync_copy(col_hbm.at[pl.ds(off, _CH)], idx)
            pltpu.sync_copy(m_hbm.at[pl.ds(off, _CH), pl.ds(0, w)], val)
            pltpu.sync_copy(val, acc.at[idx], add=True)

        return carry

    lax.fori_loop(0, (_NCHUNK + _NW - 1) // _NW, body, 0)
    plsc.subcore_barrier()

    @pl.when((sid == 0) & (cid == 0))
    def _flush0():
        pltpu.sync_copy(acc, p0_o)

    @pl.when((sid == 0) & (cid == 1))
    def _flush1():
        pltpu.sync_copy(acc, p1_o)


def _make_sc_gather():
    return functools.partial(
        pl.kernel,
        _sc_gather,
        mesh=_mesh(),
        out_type=[jax.ShapeDtypeStruct((_E, _TW), _F32),
                  jax.ShapeDtypeStruct((_E, _TW), _F32)],
        scratch_types=[pltpu.VMEM((_CH,), jnp.int32),
                       pltpu.VMEM((_CH,), jnp.int32),
                       pltpu.VMEM((_CH, _TW), _F32),
                       pltpu.VMEM((_CH, _TW), _F32),
                       pltpu.SemaphoreType.DMA,
                       pltpu.SemaphoreType.DMA],
    )()


def _make_sc_scatter(w):
    return functools.partial(
        pl.kernel,
        _sc_scatter,
        mesh=_mesh(),
        out_type=[jax.ShapeDtypeStruct((_N, w), _F32),
                  jax.ShapeDtypeStruct((_N, w), _F32)],
        scratch_types=[pltpu.VMEM((_CH,), jnp.int32),
                       pltpu.VMEM((_CH, w), _F32),
                       pltpu.VMEM_SHARED((_N, w), _F32),
                       pltpu.SemaphoreType.DMA],
    )()


def _full(shape):
    nd = len(shape)
    return pl.BlockSpec(shape, lambda i: (0,) * nd)


def kernel(x, edge_index, edge_attr, u, batch,
           e1_1_W, e1_1_b, e1_2_W, e1_2_b,
           n1_1_W, n1_1_b, n1_2_W, n1_2_b,
           n1_3_W, n1_3_b, n1_4_W, n1_4_b,
           g1_1_W, g1_1_b, g1_2_W, g1_2_b,
           e2_1_W, e2_1_b, e2_2_W, e2_2_b,
           n2_1_W, n2_1_b, n2_2_W, n2_2_b,
           n2_3_W, n2_3_b, n2_4_W, n2_4_b,
           g2_1_W, g2_1_b, g2_2_W, g2_2_b,
           cls_W, cls_b):
    n, d = x.shape
    e = edge_index.shape[1]
    g, dg = u.shape
    de = edge_attr.shape[1]
    assert (n, e, g) == (_N, _E, _G)

    row = edge_index[0]
    col = edge_index[1]
    nb_grid = n // _NBLK
    eb_grid = e // _EBLK
    batchf = batch.astype(_F32).reshape(nb_grid, 1, _NBLK)
    zeros_acc1 = jnp.zeros((_N, _MW), _F32)
    zeros_acc2 = jnp.zeros((_N, _MW2), _F32)

    def rT(w):
        return jnp.transpose(w)

    def b2d(b):
        return b.reshape(1, -1)

    w1 = e1_1_W
    t1 = pl.pallas_call(
        _proj1_body,
        grid=(nb_grid,),
        in_specs=[
            pl.BlockSpec((1, 1, _NBLK), lambda i: (i, 0, 0)),
            pl.BlockSpec((_NBLK, d), lambda i: (i, 0)),
            _full((g, dg)),
            _full((d, _H)), _full((d, _H)), _full((d, _H)),
            _full((dg, _H)),
            _full((1, _H)), _full((1, _H)),
        ],
        out_specs=pl.BlockSpec((_NBLK, _TW), lambda i: (i, 0)),
        out_shape=jax.ShapeDtypeStruct((n, _TW), _F32),
    )(batchf, x, u,
      rT(w1[:, :d]), rT(w1[:, d:2 * d]), rT(n1_1_W[:, :d]),
      rT(w1[:, 2 * d + de:]),
      b2d(e1_1_b), b2d(n1_1_b))

    g1r, g1c = _make_sc_gather()(row, col, t1)

    we2s1 = rT(e1_2_W) * _BN
    wme1 = rT(n1_1_W[:, d:])
    ea1, m1 = pl.pallas_call(
        functools.partial(_edge_body, True),
        grid=(eb_grid,),
        in_specs=[
            pl.BlockSpec((_EBLK, _TW), lambda i: (i, 0)),
            pl.BlockSpec((_EBLK, _TW), lambda i: (i, 0)),
            pl.BlockSpec((_EBLK, de), lambda i: (i, 0)),
            _full((de, _H)), _full((_H, 2 * _H)), _full((1, _H)),
            _full((1, _H)), _full((_H, _H)), _full((1, _H)),
        ],
        out_specs=[
            pl.BlockSpec((_EBLK, _H), lambda i: (i, 0)),
            pl.BlockSpec((_EBLK, _MW), lambda i: (i, 0)),
        ],
        out_shape=[jax.ShapeDtypeStruct((e, _H), _F32),
                   jax.ShapeDtypeStruct((e, _MW), _F32)],
    )(g1r, g1c, edge_attr,
      rT(e1_1_W[:, 2 * d:2 * d + de]),
      jnp.concatenate([we2s1, _dotT(we2s1, wme1)], axis=1), b2d(e1_2_b),
      _dotT(b2d(e1_2_b), wme1), rT(n1_2_W) * _BN, b2d(n1_2_b))

    p0, p1 = _make_sc_scatter(_MW)(m1, col, zeros_acc1)

    x2, nacc1, rcp = pl.pallas_call(
        _node1_body,
        grid=(nb_grid,),
        in_specs=[
            pl.BlockSpec((1, 1, _NBLK), lambda i: (i, 0, 0)),
            pl.BlockSpec((_NBLK, d), lambda i: (i, 0)),
            pl.BlockSpec((_NBLK, _MW), lambda i: (i, 0)),
            pl.BlockSpec((_NBLK, _MW), lambda i: (i, 0)),
            _full((g, dg)),
            _full((d, _H)), _full((_H, _H)), _full((dg, _H)),
            _full((1, _H)), _full((_H, _H)), _full((1, _H)),
        ],
        out_specs=[
            pl.BlockSpec((_NBLK, _H), lambda i: (i, 0)),
            _full((g, _H + 8)),
            pl.BlockSpec((_NBLK, 8), lambda i: (i, 0)),
        ],
        out_shape=[jax.ShapeDtypeStruct((n, _H), _F32),
                   jax.ShapeDtypeStruct((g, _H + 8), _F32),
                   jax.ShapeDtypeStruct((n, 8), _F32)],
    )(batchf, x, p0, p1, u,
      rT(n1_3_W[:, :d]), rT(n1_3_W[:, d:d + _H]), rT(n1_3_W[:, d + _H:]),
      b2d(n1_3_b), rT(n1_4_W) * _BN, b2d(n1_4_b))

    w2 = e2_1_W
    t2, u2 = pl.pallas_call(
        _proj2_body,
        grid=(nb_grid,),
        in_specs=[
            pl.BlockSpec((1, 1, _NBLK), lambda i: (i, 0, 0)),
            pl.BlockSpec((_NBLK, _H), lambda i: (i, 0)),
            _full((g, dg)),
            _full((g, _H + 8)),
            _full((dg, _H)), _full((_H, _H)), _full((1, _H)),
            _full((_H, _H)), _full((1, _H)),
            _full((_H, _H)), _full((_H, _H)), _full((_H, _H)),
            _full((_H, _H)),
            _full((1, _H)), _full((1, _H)),
        ],
        out_specs=[
            pl.BlockSpec((_NBLK, _TW), lambda i: (i, 0)),
            _full((g, _H)),
        ],
        out_shape=[jax.ShapeDtypeStruct((n, _TW), _F32),
                   jax.ShapeDtypeStruct((g, _H), _F32)],
    )(batchf, x2, u, nacc1,
      rT(g1_1_W[:, :dg]), rT(g1_1_W[:, dg:]), b2d(g1_1_b),
      rT(g1_2_W), b2d(g1_2_b),
      rT(w2[:, :_H]), rT(w2[:, _H:2 * _H]), rT(n2_1_W[:, :_H]),
      rT(w2[:, 3 * _H:]),
      b2d(e2_1_b), b2d(n2_1_b))

    g2r, g2c = _make_sc_gather()(row, col, t2)

    we2s2 = rT(e2_2_W) * _BN
    wme2 = rT(n2_1_W[:, _H:])
    ea2, m2 = pl.pallas_call(
        functools.partial(_edge_body, False),
        grid=(eb_grid,),
        in_specs=[
            pl.BlockSpec((_EBLK, _TW), lambda i: (i, 0)),
            pl.BlockSpec((_EBLK, _TW), lambda i: (i, 0)),
            pl.BlockSpec((_EBLK, _H), lambda i: (i, 0)),
            _full((_H, _H)), _full((_H, 2 * _H)), _full((1, _H)),
            _full((1, _H)), _full((_H, _H)), _full((1, _H)),
        ],
        out_specs=[
            pl.BlockSpec((_EBLK, _H), lambda i: (i, 0)),
            pl.BlockSpec((_EBLK, _MW2), lambda i: (i, 0)),
        ],
        out_shape=[jax.ShapeDtypeStruct((e, _H), _F32),
                   jax.ShapeDtypeStruct((e, _MW2), _F32)],
    )(g2r, g2c, ea1,
      rT(e2_1_W[:, 2 * _H:3 * _H]),
      jnp.concatenate([we2s2, _dotT(we2s2, wme2)], axis=1), b2d(e2_2_b),
      _dotT(b2d(e2_2_b), wme2), rT(n2_2_W) * _BN, b2d(n2_2_b))

    q0, q1 = _make_sc_scatter(_MW2)(m2, col, zeros_acc2)

    x3, out, nacc2 = pl.pallas_call(
        _node2_body,
        grid=(nb_grid,),
        in_specs=[
            pl.BlockSpec((1, 1, _NBLK), lambda i: (i, 0, 0)),
            pl.BlockSpec((_NBLK, _H), lambda i: (i, 0)),
            pl.BlockSpec((_NBLK, _MW2), lambda i: (i, 0)),
            pl.BlockSpec((_NBLK, _MW2), lambda i: (i, 0)),
            pl.BlockSpec((_NBLK, 8), lambda i: (i, 0)),
            _full((g, _H)),
            _full((_H, _H)), _full((_H, _H)), _full((_H, _H)),
            _full((1, _H)), _full((_H, _H)), _full((1, _H)),
            _full((_H, 20)), _full((1, 20)),
        ],
        out_specs=[
            pl.BlockSpec((_NBLK, _H), lambda i: (i, 0)),
            pl.BlockSpec((_NBLK, 20), lambda i: (i, 0)),
            _full((g, _H + 8)),
        ],
        out_shape=[jax.ShapeDtypeStruct((n, _H), _F32),
                   jax.ShapeDtypeStruct((n, 20), _F32),
                   jax.ShapeDtypeStruct((g, _H + 8), _F32)],
    )(batchf, x2, q0, q1, rcp, u2,
      rT(n2_3_W[:, :_H]), rT(n2_3_W[:, _H:2 * _H]), rT(n2_3_W[:, 2 * _H:]),
      b2d(n2_3_b), rT(n2_4_W) * _BN, b2d(n2_4_b),
      rT(cls_W), b2d(cls_b))

    u3 = pl.pallas_call(
        _gfinal_body,
        grid=(1,),
        in_specs=[
            _full((g, _H)), _full((g, _H + 8)),
            _full((_H, _H)), _full((_H, _H)), _full((1, _H)),
            _full((_H, _H)), _full((1, _H)),
        ],
        out_specs=_full((g, _H)),
        out_shape=jax.ShapeDtypeStruct((g, _H), _F32),
    )(u2, nacc2,
      rT(g2_1_W[:, :_H]), rT(g2_1_W[:, _H:]), b2d(g2_1_b),
      rT(g2_2_W), b2d(g2_2_b))

    return (out, x3, ea2, u3)
